# baseline (speedup 1.0000x reference)
import functools

import jax
import jax.numpy as jnp
from jax.experimental import pallas as pl

R = 11.0
R2T = 121.0 + 2.0 ** -17
EPS = 1e-06

N = 16384
M = 1024
D = 64
NB = 512
GRID = N // NB


def _chart_cover_kernel(z_ref, c_ref, mu_ref, var_ref,
                        dists_ref, idx_ref, masks_ref):
    mu = mu_ref[...]
    var = var_ref[...]
    z_w = (z_ref[...] - mu) / jnp.sqrt(var + EPS)
    c = c_ref[...]
    z2 = jnp.sum(z_w * z_w, axis=1, keepdims=True)
    c2 = jnp.sum(c * c, axis=1)[None, :]
    zc = jax.lax.dot_general(
        z_w, c, (((1,), (1,)), ((), ())),
        preferred_element_type=jnp.float32)
    d2 = jnp.maximum(z2 + c2 - 2.0 * zc, 0.0)
    dists = jnp.sqrt(d2)
    dists_ref[...] = dists[None, :1, :]

    dmin = jnp.min(dists, axis=1, keepdims=True)
    col = jax.lax.broadcasted_iota(jnp.int32, (NB, M), 1)
    idx = jnp.min(jnp.where(dists == dmin, col, M), axis=1)
    idx_ref[...] = idx[None, None, :]

    zc_t = jax.lax.dot_general(
        c, z_w, (((1,), (1,)), ((), ())),
        preferred_element_type=jnp.float32)
    d2_t = c2.T + z2.T - 2.0 * zc_t
    masks_ref[...] = d2_t <= R2T


@functools.partial(jax.jit, static_argnames=())
def kernel(z, centers, stats_mean, stats_var):
    dists, idx2d, masks = pl.pallas_call(
        _chart_cover_kernel,
        grid=(GRID,),
        in_specs=[
            pl.BlockSpec((NB, D), lambda i: (i, 0)),
            pl.BlockSpec((M, D), lambda i: (0, 0)),
            pl.BlockSpec((D,), lambda i: (0,)),
            pl.BlockSpec((D,), lambda i: (0,)),
        ],
        out_specs=[
            pl.BlockSpec((1, 1, M), lambda i: (i, 0, 0)),
            pl.BlockSpec((1, 1, NB), lambda i: (i, 0, 0)),
            pl.BlockSpec((M, NB), lambda i: (0, i)),
        ],
        out_shape=[
            jax.ShapeDtypeStruct((GRID, 1, M), jnp.float32),
            jax.ShapeDtypeStruct((GRID, 1, NB), jnp.int32),
            jax.ShapeDtypeStruct((M, N), jnp.bool_),
        ],
    )(z, centers, stats_mean, stats_var)
    return dists, idx2d.reshape(N), masks


# speedup vs baseline: 1.1511x; 1.1511x over previous
import functools

import jax
import jax.numpy as jnp
from jax.experimental import pallas as pl

R = 11.0
R2T = 121.0 + 2.0 ** -17
EPS = 1e-06

N = 16384
M = 1024
D = 64
NB = 512
GRID = N // NB
MG = M // 8


def _chart_cover_kernel(z_ref, c_ref, mu_ref, var_ref,
                        dists_ref, idx_ref, masks_ref):
    mu = mu_ref[...]
    var = var_ref[...]
    z_w = (z_ref[...] - mu) / jnp.sqrt(var + EPS)
    c = c_ref[...]
    z2 = jnp.sum(z_w * z_w, axis=1)[None, :]
    c2 = jnp.sum(c * c, axis=1)[:, None]
    zc_t = jax.lax.dot_general(
        c, z_w, (((1,), (1,)), ((), ())),
        preferred_element_type=jnp.float32)
    d2_t = c2 + z2 - 2.0 * zc_t
    masks_ref[...] = d2_t <= R2T

    d2c_t = jnp.maximum(d2_t, 0.0)

    d3 = d2c_t.reshape(MG, 8, NB)
    vmin8 = jnp.min(d3, axis=0)
    g3 = jax.lax.broadcasted_iota(jnp.int32, (MG, 8, NB), 0)
    gidx = jnp.min(jnp.where(d3 == vmin8[None], g3, MG), axis=0)
    row8 = gidx * 8 + jax.lax.broadcasted_iota(jnp.int32, (8, NB), 0)
    dmin = jnp.min(vmin8, axis=0, keepdims=True)
    idx = jnp.min(jnp.where(vmin8 == dmin, row8, M), axis=0)
    idx_ref[...] = idx[None, None, :]

    dt = d2c_t.T
    dists_ref[...] = dt * jax.lax.rsqrt(jnp.maximum(dt, 1e-35))


@functools.partial(jax.jit, static_argnames=())
def kernel(z, centers, stats_mean, stats_var):
    dists, idx2d, masks = pl.pallas_call(
        _chart_cover_kernel,
        grid=(GRID,),
        in_specs=[
            pl.BlockSpec((NB, D), lambda i: (i, 0)),
            pl.BlockSpec((M, D), lambda i: (0, 0)),
            pl.BlockSpec((D,), lambda i: (0,)),
            pl.BlockSpec((D,), lambda i: (0,)),
        ],
        out_specs=[
            pl.BlockSpec((NB, M), lambda i: (i, 0)),
            pl.BlockSpec((1, 1, NB), lambda i: (i, 0, 0)),
            pl.BlockSpec((M, NB), lambda i: (0, i)),
        ],
        out_shape=[
            jax.ShapeDtypeStruct((N, M), jnp.float32),
            jax.ShapeDtypeStruct((GRID, 1, NB), jnp.int32),
            jax.ShapeDtypeStruct((M, N), jnp.bool_),
        ],
    )(z, centers, stats_mean, stats_var)
    return dists, idx2d.reshape(N), masks


# speedup vs baseline: 1.1548x; 1.0032x over previous
import functools

import jax
import jax.numpy as jnp
from jax.experimental import pallas as pl
from jax.experimental.pallas import tpu as pltpu

R = 11.0
R2T = 121.0 + 2.0 ** -17
EPS = 1e-06

N = 16384
M = 1024
D = 64
NB = 512
GRID = N // NB
MG = M // 8


def _chart_cover_kernel(z_ref, c_ref, mu_ref, var_ref,
                        dists_ref, idx_ref, masks_ref):
    mu = mu_ref[...]
    var = var_ref[...]
    z_w = (z_ref[...] - mu) / jnp.sqrt(var + EPS)
    c = c_ref[...]
    z2 = jnp.sum(z_w * z_w, axis=1)[None, :]
    c2 = jnp.sum(c * c, axis=1)[:, None]
    zc_t = jax.lax.dot_general(
        c, z_w, (((1,), (1,)), ((), ())),
        preferred_element_type=jnp.float32)
    d2_t = c2 + z2 - 2.0 * zc_t
    masks_ref[...] = d2_t <= R2T

    d2c_t = jnp.maximum(d2_t, 0.0)

    d3 = d2c_t.reshape(MG, 8, NB)
    vmin8 = jnp.min(d3, axis=0)
    g3 = jax.lax.broadcasted_iota(jnp.int32, (MG, 8, NB), 0)
    gidx = jnp.min(jnp.where(d3 == vmin8[None], g3, MG), axis=0)
    row8 = gidx * 8 + jax.lax.broadcasted_iota(jnp.int32, (8, NB), 0)
    dmin = jnp.min(vmin8, axis=0, keepdims=True)
    idx = jnp.min(jnp.where(vmin8 == dmin, row8, M), axis=0)
    idx_ref[...] = idx[None, None, :]

    dt = d2c_t.T
    dists_ref[...] = dt * jax.lax.rsqrt(jnp.maximum(dt, 1e-35))


@functools.partial(jax.jit, static_argnames=())
def kernel(z, centers, stats_mean, stats_var):
    dists, idx2d, masks = pl.pallas_call(
        _chart_cover_kernel,
        grid=(GRID,),
        compiler_params=pltpu.CompilerParams(
            dimension_semantics=("parallel",)),
        in_specs=[
            pl.BlockSpec((NB, D), lambda i: (i, 0)),
            pl.BlockSpec((M, D), lambda i: (0, 0)),
            pl.BlockSpec((D,), lambda i: (0,)),
            pl.BlockSpec((D,), lambda i: (0,)),
        ],
        out_specs=[
            pl.BlockSpec((NB, M), lambda i: (i, 0)),
            pl.BlockSpec((1, 1, NB), lambda i: (i, 0, 0)),
            pl.BlockSpec((M, NB), lambda i: (0, i)),
        ],
        out_shape=[
            jax.ShapeDtypeStruct((N, M), jnp.float32),
            jax.ShapeDtypeStruct((GRID, 1, NB), jnp.int32),
            jax.ShapeDtypeStruct((M, N), jnp.bool_),
        ],
    )(z, centers, stats_mean, stats_var)
    return dists, idx2d.reshape(N), masks


# speedup vs baseline: 1.2402x; 1.0739x over previous
import functools

import jax
import jax.numpy as jnp
from jax.experimental import pallas as pl
from jax.experimental.pallas import tpu as pltpu

R = 11.0
R2T = 121.0 + 2.0 ** -17
EPS = 1e-06

N = 16384
M = 1024
D = 64
NB = 1024
GRID = N // NB
MG = M // 8


def _chart_cover_kernel(z_ref, c_ref, mu_ref, var_ref,
                        dists_ref, idx_ref, masks_ref):
    mu = mu_ref[...]
    var = var_ref[...]
    z_w = (z_ref[...] - mu) / jnp.sqrt(var + EPS)
    c = c_ref[...]
    z2 = jnp.sum(z_w * z_w, axis=1)[None, :]
    c2 = jnp.sum(c * c, axis=1)[:, None]
    zc_t = jax.lax.dot_general(
        c, z_w, (((1,), (1,)), ((), ())),
        preferred_element_type=jnp.float32)
    d2_t = c2 + z2 - 2.0 * zc_t
    masks_ref[...] = d2_t <= R2T

    d2c_t = jnp.maximum(d2_t, 0.0)

    d3 = d2c_t.reshape(MG, 8, NB)
    vmin8 = jnp.min(d3, axis=0)
    g3 = jax.lax.broadcasted_iota(jnp.int32, (MG, 8, NB), 0)
    gidx = jnp.min(jnp.where(d3 == vmin8[None], g3, MG), axis=0)
    row8 = gidx * 8 + jax.lax.broadcasted_iota(jnp.int32, (8, NB), 0)
    dmin = jnp.min(vmin8, axis=0, keepdims=True)
    idx = jnp.min(jnp.where(vmin8 == dmin, row8, M), axis=0)
    idx_ref[...] = idx[None, None, :]

    dt = d2c_t.T
    dists_ref[...] = dt * jax.lax.rsqrt(jnp.maximum(dt, 1e-35))


@functools.partial(jax.jit, static_argnames=())
def kernel(z, centers, stats_mean, stats_var):
    dists, idx2d, masks = pl.pallas_call(
        _chart_cover_kernel,
        grid=(GRID,),
        compiler_params=pltpu.CompilerParams(
            dimension_semantics=("parallel",)),
        in_specs=[
            pl.BlockSpec((NB, D), lambda i: (i, 0)),
            pl.BlockSpec((M, D), lambda i: (0, 0)),
            pl.BlockSpec((D,), lambda i: (0,)),
            pl.BlockSpec((D,), lambda i: (0,)),
        ],
        out_specs=[
            pl.BlockSpec((NB, M), lambda i: (i, 0)),
            pl.BlockSpec((1, 1, NB), lambda i: (i, 0, 0)),
            pl.BlockSpec((M, NB), lambda i: (0, i)),
        ],
        out_shape=[
            jax.ShapeDtypeStruct((N, M), jnp.float32),
            jax.ShapeDtypeStruct((GRID, 1, NB), jnp.int32),
            jax.ShapeDtypeStruct((M, N), jnp.bool_),
        ],
    )(z, centers, stats_mean, stats_var)
    return dists, idx2d.reshape(N), masks
